# baseline (speedup 1.0000x reference)
import functools

import jax
import jax.numpy as jnp
from jax import lax
from jax.experimental import pallas as pl
from jax.experimental.pallas import tpu as pltpu
from jax.experimental.pallas import tpu_sc as plsc

N = 10000
E = 320000
D_IN = 128
D_HID = 75
D_OUT = 128

NC = 2
NS = 16
NW = NC * NS
K = 128
CT = 160
C0 = 80
C1 = CT - C0
E_PAD = NS * CT * K
C = E_PAD // (NW * K)
N_PAD = 10240
RPT = N_PAD // NS
DH_PAD = 80
BLK = 512

_mesh = plsc.VectorSubcoreMesh(
    core_axis_name="c", subcore_axis_name="s", num_cores=NC, num_subcores=NS
)


@functools.partial(
    pl.kernel,
    out_type=jax.ShapeDtypeStruct((NC * N_PAD,), jnp.float32),
    mesh=_mesh,
    scratch_types=[
        pltpu.VMEM((C, K), jnp.int32),
        pltpu.VMEM((K,), jnp.float32),
        pltpu.VMEM((RPT,), jnp.float32),
        pltpu.VMEM_SHARED((N_PAD,), jnp.float32),
    ],
)
def _sc_deg(dst_hbm, deg_hbm, dstv, onesv, zv, degs):
    ci = lax.axis_index("c")
    si = lax.axis_index("s")
    wid = si * NC + ci
    for i in range(K // 16):
        onesv[pl.ds(i * 16, 16)] = jnp.full((16,), 1.0, jnp.float32)
    for i in range(RPT // 16):
        zv[pl.ds(i * 16, 16)] = jnp.zeros((16,), jnp.float32)
    pltpu.sync_copy(zv, degs.at[pl.ds(si * RPT, RPT)])
    pltpu.sync_copy(dst_hbm.at[wid], dstv)
    plsc.subcore_barrier()

    def body(j, carry):
        pltpu.sync_copy(onesv, degs.at[dstv.at[j]], add=True)
        return carry

    lax.fori_loop(0, C, body, 0)
    plsc.subcore_barrier()
    pltpu.sync_copy(
        degs.at[pl.ds(si * RPT, RPT)],
        deg_hbm.at[pl.ds(ci * N_PAD + si * RPT, RPT)],
    )


def _make_sc_agg(D):
    @functools.partial(
        pl.kernel,
        out_type=jax.ShapeDtypeStruct((NC * N_PAD, D), jnp.float32),
        mesh=_mesh,
        scratch_types=[
            pltpu.VMEM((C0, K), jnp.int32),
            pltpu.VMEM((1, K), jnp.int32),
            pltpu.VMEM((1, K), jnp.int32),
            pltpu.VMEM((K, D), jnp.float32),
            pltpu.VMEM((K, D), jnp.float32),
            pltpu.VMEM((8, D), jnp.float32),
            pltpu.VMEM_SHARED((N_PAD, D), jnp.float32),
            pltpu.SemaphoreType.DMA,
            pltpu.SemaphoreType.DMA,
        ],
        compiler_params=pltpu.CompilerParams(use_tc_tiling_on_sc=False),
    )
    def agg(g_hbm, src_hbm, dst_hbm, out_hbm, srcv, dstb0, dstb1, rows0,
            rows1, zv, accs, gsem, dsem):
        ci = lax.axis_index("c")
        si = lax.axis_index("s")
        cstart = jnp.where(ci == 0, 0, C0)
        npairs = jnp.where(ci == 0, C0 // 2, C1 // 2)

        for r in range(8):
            for c5 in range(D // 16):
                zv[r, pl.ds(c5 * 16, 16)] = jnp.zeros((16,), jnp.float32)
        for t in range(RPT // 8):
            pltpu.sync_copy(zv, accs.at[pl.ds(si * RPT + t * 8, 8)])

        @pl.when(ci == 0)
        def _():
            pltpu.sync_copy(src_hbm.at[si, pl.ds(0, C0)],
                            srcv.at[pl.ds(0, C0)])

        @pl.when(ci != 0)
        def _():
            pltpu.sync_copy(src_hbm.at[si, pl.ds(C0, C1)],
                            srcv.at[pl.ds(0, C1)])

        plsc.subcore_barrier()

        pltpu.async_copy(g_hbm.at[srcv.at[0]], rows0, gsem)
        pltpu.async_copy(dst_hbm.at[si, cstart], dstb0.at[0], dsem)

        def body(i, carry):
            j = i * 2
            pltpu.make_async_copy(g_hbm.at[srcv.at[j]], rows0, gsem).wait()
            pltpu.async_copy(g_hbm.at[srcv.at[j + 1]], rows1, gsem)
            pltpu.make_async_copy(
                dst_hbm.at[si, cstart + j], dstb0.at[0], dsem).wait()
            pltpu.async_copy(dst_hbm.at[si, cstart + j + 1], dstb1.at[0], dsem)
            pltpu.sync_copy(rows0, accs.at[dstb0.at[0]], add=True)
            pltpu.make_async_copy(g_hbm.at[srcv.at[j + 1]], rows1, gsem).wait()
            pltpu.make_async_copy(
                dst_hbm.at[si, cstart + j + 1], dstb1.at[0], dsem).wait()

            @pl.when(i + 1 < npairs)
            def _():
                pltpu.async_copy(g_hbm.at[srcv.at[j + 2]], rows0, gsem)
                pltpu.async_copy(
                    dst_hbm.at[si, cstart + j + 2], dstb0.at[0], dsem)

            pltpu.sync_copy(rows1, accs.at[dstb1.at[0]], add=True)
            return carry

        lax.fori_loop(0, npairs, body, 0)
        plsc.subcore_barrier()
        base = si * RPT
        pltpu.sync_copy(
            accs.at[pl.ds(base, RPT)],
            out_hbm.at[pl.ds(ci * N_PAD + base, RPT)],
        )

    return agg


_sc_agg_hid = _make_sc_agg(DH_PAD)
_sc_agg_out = _make_sc_agg(D_OUT)


def _tc_lin1_body(x_ref, w_ref, deg_ref, g_ref):
    dg = deg_ref[:, 0] + deg_ref[:, 1] + 1.0
    dinv = lax.rsqrt(dg)[:, None]
    h = jnp.dot(x_ref[:], w_ref[:], preferred_element_type=jnp.float32)
    g_ref[:] = h * dinv


def _tc_mid_body(a0_ref, a1_ref, g1_ref, deg_ref, b1_ref, w2_ref, g2_ref):
    dg = deg_ref[:, 0] + deg_ref[:, 1] + 1.0
    dinv = lax.rsqrt(dg)[:, None]
    z = dinv * (a0_ref[:] + a1_ref[:] + g1_ref[:]) + b1_ref[:]
    r = jnp.maximum(z, 0.0)
    h2 = jnp.dot(r, w2_ref[:], preferred_element_type=jnp.float32)
    g2_ref[:] = h2 * dinv


def _tc_out_body(a0_ref, a1_ref, g2_ref, deg_ref, b2_ref, out_ref):
    dg = deg_ref[:, 0] + deg_ref[:, 1] + 1.0
    dinv = lax.rsqrt(dg)[:, None]
    z = dinv * (a0_ref[:] + a1_ref[:] + g2_ref[:]) + b2_ref[:]
    m = jnp.max(z, axis=1, keepdims=True)
    lse = m + jnp.log(jnp.sum(jnp.exp(z - m), axis=1, keepdims=True))
    out_ref[:] = z - lse


def _tc_lin1(x_pad, w1p, deg2t):
    return pl.pallas_call(
        _tc_lin1_body,
        grid=(N_PAD // BLK,),
        in_specs=[
            pl.BlockSpec((BLK, D_IN), lambda i: (i, 0)),
            pl.BlockSpec((D_IN, DH_PAD), lambda i: (0, 0)),
            pl.BlockSpec((BLK, NC), lambda i: (i, 0)),
        ],
        out_specs=pl.BlockSpec((BLK, DH_PAD), lambda i: (i, 0)),
        out_shape=jax.ShapeDtypeStruct((N_PAD, DH_PAD), jnp.float32),
    )(x_pad, w1p, deg2t)


def _tc_mid(a0, a1, g1, deg2t, b1p, w2p):
    return pl.pallas_call(
        _tc_mid_body,
        grid=(N_PAD // BLK,),
        in_specs=[
            pl.BlockSpec((BLK, DH_PAD), lambda i: (i, 0)),
            pl.BlockSpec((BLK, DH_PAD), lambda i: (i, 0)),
            pl.BlockSpec((BLK, DH_PAD), lambda i: (i, 0)),
            pl.BlockSpec((BLK, NC), lambda i: (i, 0)),
            pl.BlockSpec((1, DH_PAD), lambda i: (0, 0)),
            pl.BlockSpec((DH_PAD, D_OUT), lambda i: (0, 0)),
        ],
        out_specs=pl.BlockSpec((BLK, D_OUT), lambda i: (i, 0)),
        out_shape=jax.ShapeDtypeStruct((N_PAD, D_OUT), jnp.float32),
    )(a0, a1, g1, deg2t, b1p, w2p)


def _tc_out(a0, a1, g2, deg2t, b2r):
    return pl.pallas_call(
        _tc_out_body,
        grid=(N_PAD // BLK,),
        in_specs=[
            pl.BlockSpec((BLK, D_OUT), lambda i: (i, 0)),
            pl.BlockSpec((BLK, D_OUT), lambda i: (i, 0)),
            pl.BlockSpec((BLK, D_OUT), lambda i: (i, 0)),
            pl.BlockSpec((BLK, NC), lambda i: (i, 0)),
            pl.BlockSpec((1, D_OUT), lambda i: (0, 0)),
        ],
        out_specs=pl.BlockSpec((BLK, D_OUT), lambda i: (i, 0)),
        out_shape=jax.ShapeDtypeStruct((N_PAD, D_OUT), jnp.float32),
    )(a0, a1, g2, deg2t, b2r)


@jax.jit
def _run(x, edge_index, W1, b1, W2, b2):
    x_pad = jnp.zeros((N_PAD, D_IN), jnp.float32).at[:N].set(x)
    pad = jnp.full((E_PAD - E,), N, jnp.int32)
    src_flat = jnp.concatenate([edge_index[0], pad])
    dst_flat = jnp.concatenate([edge_index[1], pad])
    src = src_flat.reshape(NS, CT, K)
    dst = dst_flat.reshape(NS, CT, K)
    dst_deg = dst_flat.reshape(NW, C, K)
    w1p = jnp.zeros((D_IN, DH_PAD), jnp.float32).at[:, :D_HID].set(W1)
    b1p = jnp.zeros((1, DH_PAD), jnp.float32).at[0, :D_HID].set(b1)
    w2p = jnp.zeros((DH_PAD, D_OUT), jnp.float32).at[:D_HID].set(W2)
    b2r = b2.reshape(1, D_OUT)

    degf = _sc_deg(dst_deg)
    deg2t = degf.reshape(NC, N_PAD).T

    g1 = _tc_lin1(x_pad, w1p, deg2t)
    acc1 = _sc_agg_hid(g1, src, dst)
    g2 = _tc_mid(acc1[:N_PAD], acc1[N_PAD:], g1, deg2t, b1p, w2p)
    acc2 = _sc_agg_out(g2, src, dst)
    out = _tc_out(acc2[:N_PAD], acc2[N_PAD:], g2, deg2t, b2r)
    return out[:N]


def kernel(x, edge_index, W1, b1, W2, b2):
    return _run(x, edge_index, W1, b1, W2, b2)


# speedup vs baseline: 1.1386x; 1.1386x over previous
import functools

import jax
import jax.numpy as jnp
from jax import lax
from jax.experimental import pallas as pl
from jax.experimental.pallas import tpu as pltpu
from jax.experimental.pallas import tpu_sc as plsc

N = 10000
E = 320000
D_IN = 128
D_HID = 75
D_OUT = 128

NC = 2
NS = 16
NW = NC * NS
K = 128
CT = 160
C0 = 111
C1 = CT - C0
E_PAD = NS * CT * K
C = E_PAD // (NW * K)
N_PAD = 10240
RPT = N_PAD // NS
DH_PAD = 80
BLK = 512

_mesh = plsc.VectorSubcoreMesh(
    core_axis_name="c", subcore_axis_name="s", num_cores=NC, num_subcores=NS
)


@functools.partial(
    pl.kernel,
    out_type=jax.ShapeDtypeStruct((NC * N_PAD,), jnp.float32),
    mesh=_mesh,
    scratch_types=[
        pltpu.VMEM((C, K), jnp.int32),
        pltpu.VMEM((K,), jnp.float32),
        pltpu.VMEM((RPT,), jnp.float32),
        pltpu.VMEM_SHARED((N_PAD,), jnp.float32),
    ],
)
def _sc_deg(dst_hbm, deg_hbm, dstv, onesv, zv, degs):
    ci = lax.axis_index("c")
    si = lax.axis_index("s")
    wid = si * NC + ci
    for i in range(K // 16):
        onesv[pl.ds(i * 16, 16)] = jnp.full((16,), 1.0, jnp.float32)
    for i in range(RPT // 16):
        zv[pl.ds(i * 16, 16)] = jnp.zeros((16,), jnp.float32)
    pltpu.sync_copy(zv, degs.at[pl.ds(si * RPT, RPT)])
    pltpu.sync_copy(dst_hbm.at[wid], dstv)
    plsc.subcore_barrier()

    def body(j, carry):
        pltpu.sync_copy(onesv, degs.at[dstv.at[j]], add=True)
        return carry

    lax.fori_loop(0, C, body, 0)
    plsc.subcore_barrier()
    pltpu.sync_copy(
        degs.at[pl.ds(si * RPT, RPT)],
        deg_hbm.at[pl.ds(ci * N_PAD + si * RPT, RPT)],
    )


def _make_sc_agg(D):
    @functools.partial(
        pl.kernel,
        out_type=jax.ShapeDtypeStruct((NC * N_PAD, D), jnp.float32),
        mesh=_mesh,
        scratch_types=[
            pltpu.VMEM((C0, K), jnp.int32),
            pltpu.VMEM((1, K), jnp.int32),
            pltpu.VMEM((1, K), jnp.int32),
            pltpu.VMEM((K, D), jnp.float32),
            pltpu.VMEM((K, D), jnp.float32),
            pltpu.VMEM((8, D), jnp.float32),
            pltpu.VMEM_SHARED((N_PAD, D), jnp.float32),
            pltpu.SemaphoreType.DMA,
            pltpu.SemaphoreType.DMA,
        ],
        compiler_params=pltpu.CompilerParams(use_tc_tiling_on_sc=False),
    )
    def agg(g_hbm, src_hbm, dst_hbm, out_hbm, srcv, dstb0, dstb1, rows0,
            rows1, zv, accs, gsem, dsem):
        ci = lax.axis_index("c")
        si = lax.axis_index("s")
        cstart = jnp.where(ci == 0, 0, C0)
        npairs = jnp.where(ci == 0, (C0 - 1) // 2, (C1 - 1) // 2)

        for r in range(8):
            for c5 in range(D // 16):
                zv[r, pl.ds(c5 * 16, 16)] = jnp.zeros((16,), jnp.float32)
        for t in range(RPT // 8):
            pltpu.sync_copy(zv, accs.at[pl.ds(si * RPT + t * 8, 8)])

        @pl.when(ci == 0)
        def _():
            pltpu.sync_copy(src_hbm.at[si, pl.ds(0, C0)],
                            srcv.at[pl.ds(0, C0)])

        @pl.when(ci != 0)
        def _():
            pltpu.sync_copy(src_hbm.at[si, pl.ds(C0, C1)],
                            srcv.at[pl.ds(0, C1)])

        plsc.subcore_barrier()

        pltpu.async_copy(g_hbm.at[srcv.at[0]], rows0, gsem)
        pltpu.async_copy(dst_hbm.at[si, cstart], dstb0.at[0], dsem)

        def body(i, carry):
            j = i * 2
            pltpu.make_async_copy(g_hbm.at[srcv.at[j]], rows0, gsem).wait()
            pltpu.async_copy(g_hbm.at[srcv.at[j + 1]], rows1, gsem)
            pltpu.make_async_copy(
                dst_hbm.at[si, cstart + j], dstb0.at[0], dsem).wait()
            pltpu.async_copy(dst_hbm.at[si, cstart + j + 1], dstb1.at[0], dsem)
            pltpu.sync_copy(rows0, accs.at[dstb0.at[0]], add=True)
            pltpu.make_async_copy(g_hbm.at[srcv.at[j + 1]], rows1, gsem).wait()
            pltpu.async_copy(g_hbm.at[srcv.at[j + 2]], rows0, gsem)
            pltpu.make_async_copy(
                dst_hbm.at[si, cstart + j + 1], dstb1.at[0], dsem).wait()
            pltpu.async_copy(dst_hbm.at[si, cstart + j + 2], dstb0.at[0], dsem)
            pltpu.sync_copy(rows1, accs.at[dstb1.at[0]], add=True)
            return carry

        lax.fori_loop(0, npairs, body, 0)
        jlast = jnp.where(ci == 0, C0 - 1, C1 - 1)
        pltpu.make_async_copy(g_hbm.at[srcv.at[jlast]], rows0, gsem).wait()
        pltpu.make_async_copy(
            dst_hbm.at[si, cstart + jlast], dstb0.at[0], dsem).wait()
        pltpu.sync_copy(rows0, accs.at[dstb0.at[0]], add=True)
        plsc.subcore_barrier()
        base = si * RPT
        pltpu.sync_copy(
            accs.at[pl.ds(base, RPT)],
            out_hbm.at[pl.ds(ci * N_PAD + base, RPT)],
        )

    return agg


_sc_agg_hid = _make_sc_agg(DH_PAD)
_sc_agg_out = _make_sc_agg(D_OUT)


def _tc_lin1_body(x_ref, w_ref, deg_ref, g_ref):
    dg = deg_ref[:, 0] + deg_ref[:, 1] + 1.0
    dinv = lax.rsqrt(dg)[:, None]
    h = jnp.dot(x_ref[:], w_ref[:], preferred_element_type=jnp.float32)
    g_ref[:] = h * dinv


def _tc_mid_body(a0_ref, a1_ref, g1_ref, deg_ref, b1_ref, w2_ref, g2_ref):
    dg = deg_ref[:, 0] + deg_ref[:, 1] + 1.0
    dinv = lax.rsqrt(dg)[:, None]
    z = dinv * (a0_ref[:] + a1_ref[:] + g1_ref[:]) + b1_ref[:]
    r = jnp.maximum(z, 0.0)
    h2 = jnp.dot(r, w2_ref[:], preferred_element_type=jnp.float32)
    g2_ref[:] = h2 * dinv


def _tc_out_body(a0_ref, a1_ref, g2_ref, deg_ref, b2_ref, out_ref):
    dg = deg_ref[:, 0] + deg_ref[:, 1] + 1.0
    dinv = lax.rsqrt(dg)[:, None]
    z = dinv * (a0_ref[:] + a1_ref[:] + g2_ref[:]) + b2_ref[:]
    m = jnp.max(z, axis=1, keepdims=True)
    lse = m + jnp.log(jnp.sum(jnp.exp(z - m), axis=1, keepdims=True))
    out_ref[:] = z - lse


def _tc_lin1(x_pad, w1p, deg2t):
    return pl.pallas_call(
        _tc_lin1_body,
        grid=(N_PAD // BLK,),
        in_specs=[
            pl.BlockSpec((BLK, D_IN), lambda i: (i, 0)),
            pl.BlockSpec((D_IN, DH_PAD), lambda i: (0, 0)),
            pl.BlockSpec((BLK, NC), lambda i: (i, 0)),
        ],
        out_specs=pl.BlockSpec((BLK, DH_PAD), lambda i: (i, 0)),
        out_shape=jax.ShapeDtypeStruct((N_PAD, DH_PAD), jnp.float32),
    )(x_pad, w1p, deg2t)


def _tc_mid(a0, a1, g1, deg2t, b1p, w2p):
    return pl.pallas_call(
        _tc_mid_body,
        grid=(N_PAD // BLK,),
        in_specs=[
            pl.BlockSpec((BLK, DH_PAD), lambda i: (i, 0)),
            pl.BlockSpec((BLK, DH_PAD), lambda i: (i, 0)),
            pl.BlockSpec((BLK, DH_PAD), lambda i: (i, 0)),
            pl.BlockSpec((BLK, NC), lambda i: (i, 0)),
            pl.BlockSpec((1, DH_PAD), lambda i: (0, 0)),
            pl.BlockSpec((DH_PAD, D_OUT), lambda i: (0, 0)),
        ],
        out_specs=pl.BlockSpec((BLK, D_OUT), lambda i: (i, 0)),
        out_shape=jax.ShapeDtypeStruct((N_PAD, D_OUT), jnp.float32),
    )(a0, a1, g1, deg2t, b1p, w2p)


def _tc_out(a0, a1, g2, deg2t, b2r):
    return pl.pallas_call(
        _tc_out_body,
        grid=(N_PAD // BLK,),
        in_specs=[
            pl.BlockSpec((BLK, D_OUT), lambda i: (i, 0)),
            pl.BlockSpec((BLK, D_OUT), lambda i: (i, 0)),
            pl.BlockSpec((BLK, D_OUT), lambda i: (i, 0)),
            pl.BlockSpec((BLK, NC), lambda i: (i, 0)),
            pl.BlockSpec((1, D_OUT), lambda i: (0, 0)),
        ],
        out_specs=pl.BlockSpec((BLK, D_OUT), lambda i: (i, 0)),
        out_shape=jax.ShapeDtypeStruct((N_PAD, D_OUT), jnp.float32),
    )(a0, a1, g2, deg2t, b2r)


@jax.jit
def _run(x, edge_index, W1, b1, W2, b2):
    x_pad = jnp.zeros((N_PAD, D_IN), jnp.float32).at[:N].set(x)
    pad = jnp.full((E_PAD - E,), N, jnp.int32)
    src_flat = jnp.concatenate([edge_index[0], pad])
    dst_flat = jnp.concatenate([edge_index[1], pad])
    src = src_flat.reshape(NS, CT, K)
    dst = dst_flat.reshape(NS, CT, K)
    dst_deg = dst_flat.reshape(NW, C, K)
    w1p = jnp.zeros((D_IN, DH_PAD), jnp.float32).at[:, :D_HID].set(W1)
    b1p = jnp.zeros((1, DH_PAD), jnp.float32).at[0, :D_HID].set(b1)
    w2p = jnp.zeros((DH_PAD, D_OUT), jnp.float32).at[:D_HID].set(W2)
    b2r = b2.reshape(1, D_OUT)

    degf = _sc_deg(dst_deg)
    deg2t = degf.reshape(NC, N_PAD).T

    g1 = _tc_lin1(x_pad, w1p, deg2t)
    acc1 = _sc_agg_hid(g1, src, dst)
    g2 = _tc_mid(acc1[:N_PAD], acc1[N_PAD:], g1, deg2t, b1p, w2p)
    acc2 = _sc_agg_out(g2, src, dst)
    out = _tc_out(acc2[:N_PAD], acc2[N_PAD:], g2, deg2t, b2r)
    return out[:N]


def kernel(x, edge_index, W1, b1, W2, b2):
    return _run(x, edge_index, W1, b1, W2, b2)


# speedup vs baseline: 1.4183x; 1.2457x over previous
import functools

import jax
import jax.numpy as jnp
from jax import lax
from jax.experimental import pallas as pl
from jax.experimental.pallas import tpu as pltpu
from jax.experimental.pallas import tpu_sc as plsc

N = 10000
E = 320000
D_IN = 128
D_HID = 75
D_OUT = 128

NC = 2
NS = 16
NW = NC * NS
K = 128
C = -(-E // (NW * K))
E_PAD = NW * C * K
N_PAD = 10240
RPT = N_PAD // NS
DH_PAD = 80
BLK = 512

_mesh = plsc.VectorSubcoreMesh(
    core_axis_name="c", subcore_axis_name="s", num_cores=NC, num_subcores=NS
)


@functools.partial(
    pl.kernel,
    out_type=jax.ShapeDtypeStruct((NC * N_PAD,), jnp.float32),
    mesh=_mesh,
    scratch_types=[
        pltpu.VMEM((C, K), jnp.int32),
        pltpu.VMEM((K,), jnp.float32),
        pltpu.VMEM((RPT,), jnp.float32),
        pltpu.VMEM_SHARED((N_PAD,), jnp.float32),
    ],
)
def _sc_deg(dst_hbm, deg_hbm, dstv, onesv, zv, degs):
    ci = lax.axis_index("c")
    si = lax.axis_index("s")
    wid = si * NC + ci
    for i in range(K // 16):
        onesv[pl.ds(i * 16, 16)] = jnp.full((16,), 1.0, jnp.float32)
    for i in range(RPT // 16):
        zv[pl.ds(i * 16, 16)] = jnp.zeros((16,), jnp.float32)
    pltpu.sync_copy(zv, degs.at[pl.ds(si * RPT, RPT)])
    pltpu.sync_copy(dst_hbm.at[wid], dstv)
    plsc.subcore_barrier()

    def body(j, carry):
        pltpu.sync_copy(onesv, degs.at[dstv.at[j]], add=True)
        return carry

    lax.fori_loop(0, C, body, 0)
    plsc.subcore_barrier()
    pltpu.sync_copy(
        degs.at[pl.ds(si * RPT, RPT)],
        deg_hbm.at[pl.ds(ci * N_PAD + si * RPT, RPT)],
    )


def _make_sc_agg(D):
    @functools.partial(
        pl.kernel,
        out_type=jax.ShapeDtypeStruct((NC * N_PAD, D), jnp.float32),
        mesh=_mesh,
        scratch_types=[
            pltpu.VMEM((C, K), jnp.int32),
            pltpu.VMEM((1, K), jnp.int32),
            pltpu.VMEM((1, K), jnp.int32),
            pltpu.VMEM((K, D), jnp.float32),
            pltpu.VMEM((K, D), jnp.float32),
            pltpu.VMEM((16, D), jnp.float32),
            pltpu.VMEM_SHARED((N_PAD, D), jnp.float32),
            pltpu.SemaphoreType.DMA,
            pltpu.SemaphoreType.DMA,
        ],
        compiler_params=pltpu.CompilerParams(use_tc_tiling_on_sc=False),
    )
    def agg(g_hbm, src_hbm, dst_hbm, out_hbm, srcv, dstb0, dstb1, rows0,
            rows1, zv, accs, gsem, dsem):
        ci = lax.axis_index("c")
        si = lax.axis_index("s")
        wid = si * NC + ci
        for r in range(16):
            for c5 in range(D // 16):
                zv[r, pl.ds(c5 * 16, 16)] = jnp.zeros((16,), jnp.float32)
        for t in range(RPT // 16):
            pltpu.sync_copy(zv, accs.at[pl.ds(si * RPT + t * 16, 16)])
        pltpu.sync_copy(src_hbm.at[wid], srcv)
        plsc.subcore_barrier()

        pltpu.async_copy(g_hbm.at[srcv.at[0]], rows0, gsem)
        pltpu.async_copy(dst_hbm.at[wid, 0], dstb0.at[0], dsem)

        def body(i, carry):
            j = i * 2
            pltpu.make_async_copy(g_hbm.at[srcv.at[j]], rows0, gsem).wait()
            pltpu.async_copy(g_hbm.at[srcv.at[j + 1]], rows1, gsem)
            pltpu.make_async_copy(dst_hbm.at[wid, j], dstb0.at[0], dsem).wait()
            pltpu.async_copy(dst_hbm.at[wid, j + 1], dstb1.at[0], dsem)
            pltpu.sync_copy(rows0, accs.at[dstb0.at[0]], add=True)
            pltpu.make_async_copy(g_hbm.at[srcv.at[j + 1]], rows1, gsem).wait()
            pltpu.async_copy(g_hbm.at[srcv.at[j + 2]], rows0, gsem)
            pltpu.make_async_copy(dst_hbm.at[wid, j + 1], dstb1.at[0], dsem).wait()
            pltpu.async_copy(dst_hbm.at[wid, j + 2], dstb0.at[0], dsem)
            pltpu.sync_copy(rows1, accs.at[dstb1.at[0]], add=True)
            return carry

        lax.fori_loop(0, (C - 1) // 2, body, 0)
        pltpu.make_async_copy(g_hbm.at[srcv.at[C - 1]], rows0, gsem).wait()
        pltpu.make_async_copy(dst_hbm.at[wid, C - 1], dstb0.at[0], dsem).wait()
        pltpu.sync_copy(rows0, accs.at[dstb0.at[0]], add=True)
        plsc.subcore_barrier()
        base = si * RPT
        pltpu.sync_copy(
            accs.at[pl.ds(base, RPT)],
            out_hbm.at[pl.ds(ci * N_PAD + base, RPT)],
        )

    return agg


_sc_agg_hid = _make_sc_agg(DH_PAD)
_sc_agg_out = _make_sc_agg(D_OUT)


def _tc_lin1_body(x_ref, w_ref, deg_ref, g_ref):
    dg = deg_ref[:, 0] + deg_ref[:, 1] + 1.0
    dinv = lax.rsqrt(dg)[:, None]
    h = jnp.dot(x_ref[:], w_ref[:], preferred_element_type=jnp.float32)
    g_ref[:] = h * dinv


def _tc_mid_body(a0_ref, a1_ref, g1_ref, deg_ref, b1_ref, w2_ref, g2_ref):
    dg = deg_ref[:, 0] + deg_ref[:, 1] + 1.0
    dinv = lax.rsqrt(dg)[:, None]
    z = dinv * (a0_ref[:] + a1_ref[:] + g1_ref[:]) + b1_ref[:]
    r = jnp.maximum(z, 0.0)
    h2 = jnp.dot(r, w2_ref[:], preferred_element_type=jnp.float32)
    g2_ref[:] = h2 * dinv


def _tc_out_body(a0_ref, a1_ref, g2_ref, deg_ref, b2_ref, out_ref):
    dg = deg_ref[:, 0] + deg_ref[:, 1] + 1.0
    dinv = lax.rsqrt(dg)[:, None]
    z = dinv * (a0_ref[:] + a1_ref[:] + g2_ref[:]) + b2_ref[:]
    m = jnp.max(z, axis=1, keepdims=True)
    lse = m + jnp.log(jnp.sum(jnp.exp(z - m), axis=1, keepdims=True))
    out_ref[:] = z - lse


def _tc_lin1(x_pad, w1p, deg2t):
    return pl.pallas_call(
        _tc_lin1_body,
        grid=(N_PAD // BLK,),
        in_specs=[
            pl.BlockSpec((BLK, D_IN), lambda i: (i, 0)),
            pl.BlockSpec((D_IN, DH_PAD), lambda i: (0, 0)),
            pl.BlockSpec((BLK, NC), lambda i: (i, 0)),
        ],
        out_specs=pl.BlockSpec((BLK, DH_PAD), lambda i: (i, 0)),
        out_shape=jax.ShapeDtypeStruct((N_PAD, DH_PAD), jnp.float32),
    )(x_pad, w1p, deg2t)


def _tc_mid(a0, a1, g1, deg2t, b1p, w2p):
    return pl.pallas_call(
        _tc_mid_body,
        grid=(N_PAD // BLK,),
        in_specs=[
            pl.BlockSpec((BLK, DH_PAD), lambda i: (i, 0)),
            pl.BlockSpec((BLK, DH_PAD), lambda i: (i, 0)),
            pl.BlockSpec((BLK, DH_PAD), lambda i: (i, 0)),
            pl.BlockSpec((BLK, NC), lambda i: (i, 0)),
            pl.BlockSpec((1, DH_PAD), lambda i: (0, 0)),
            pl.BlockSpec((DH_PAD, D_OUT), lambda i: (0, 0)),
        ],
        out_specs=pl.BlockSpec((BLK, D_OUT), lambda i: (i, 0)),
        out_shape=jax.ShapeDtypeStruct((N_PAD, D_OUT), jnp.float32),
    )(a0, a1, g1, deg2t, b1p, w2p)


def _tc_out(a0, a1, g2, deg2t, b2r):
    return pl.pallas_call(
        _tc_out_body,
        grid=(N_PAD // BLK,),
        in_specs=[
            pl.BlockSpec((BLK, D_OUT), lambda i: (i, 0)),
            pl.BlockSpec((BLK, D_OUT), lambda i: (i, 0)),
            pl.BlockSpec((BLK, D_OUT), lambda i: (i, 0)),
            pl.BlockSpec((BLK, NC), lambda i: (i, 0)),
            pl.BlockSpec((1, D_OUT), lambda i: (0, 0)),
        ],
        out_specs=pl.BlockSpec((BLK, D_OUT), lambda i: (i, 0)),
        out_shape=jax.ShapeDtypeStruct((N_PAD, D_OUT), jnp.float32),
    )(a0, a1, g2, deg2t, b2r)


@jax.jit
def _run(x, edge_index, W1, b1, W2, b2):
    x_pad = jnp.zeros((N_PAD, D_IN), jnp.float32).at[:N].set(x)
    pad = jnp.full((E_PAD - E,), N, jnp.int32)
    src = jnp.concatenate([edge_index[0], pad]).reshape(NW, C, K)
    dst = jnp.concatenate([edge_index[1], pad]).reshape(NW, C, K)
    w1p = jnp.zeros((D_IN, DH_PAD), jnp.float32).at[:, :D_HID].set(W1)
    b1p = jnp.zeros((1, DH_PAD), jnp.float32).at[0, :D_HID].set(b1)
    w2p = jnp.zeros((DH_PAD, D_OUT), jnp.float32).at[:D_HID].set(W2)
    b2r = b2.reshape(1, D_OUT)

    degf = _sc_deg(dst)
    deg2t = degf.reshape(NC, N_PAD).T

    g1 = _tc_lin1(x_pad, w1p, deg2t)
    acc1 = _sc_agg_hid(g1, src, dst)
    g2 = _tc_mid(acc1[:N_PAD], acc1[N_PAD:], g1, deg2t, b1p, w2p)
    acc2 = _sc_agg_out(g2, src, dst)
    out = _tc_out(acc2[:N_PAD], acc2[N_PAD:], g2, deg2t, b2r)
    return out[:N]


def kernel(x, edge_index, W1, b1, W2, b2):
    return _run(x, edge_index, W1, b1, W2, b2)


# speedup vs baseline: 2.2823x; 1.6092x over previous
import functools

import jax
import jax.numpy as jnp
from jax import lax
from jax.experimental import pallas as pl
from jax.experimental.pallas import tpu as pltpu
from jax.experimental.pallas import tpu_sc as plsc

N = 10000
E = 320000
D_IN = 128
D_HID = 75
D_OUT = 128

NC = 2
NS = 16
NW = NC * NS
K = 128
C = -(-E // (NW * K))
E_PAD = NW * C * K
CTA = 159
E_PAD_A = NS * CTA * K
N_PAD = 10240
RPT = N_PAD // NS
DH_PAD = 80
BLK = 512

_mesh = plsc.VectorSubcoreMesh(
    core_axis_name="c", subcore_axis_name="s", num_cores=NC, num_subcores=NS
)


@functools.partial(
    pl.kernel,
    out_type=jax.ShapeDtypeStruct((NC * N_PAD,), jnp.float32),
    mesh=_mesh,
    scratch_types=[
        pltpu.VMEM((C, K), jnp.int32),
        pltpu.VMEM((K,), jnp.float32),
        pltpu.VMEM((RPT,), jnp.float32),
        pltpu.VMEM_SHARED((N_PAD,), jnp.float32),
    ],
)
def _sc_deg(dst_hbm, deg_hbm, dstv, onesv, zv, degs):
    ci = lax.axis_index("c")
    si = lax.axis_index("s")
    wid = si * NC + ci
    for i in range(K // 16):
        onesv[pl.ds(i * 16, 16)] = jnp.full((16,), 1.0, jnp.float32)
    for i in range(RPT // 16):
        zv[pl.ds(i * 16, 16)] = jnp.zeros((16,), jnp.float32)
    pltpu.sync_copy(zv, degs.at[pl.ds(si * RPT, RPT)])
    pltpu.sync_copy(dst_hbm.at[wid], dstv)
    plsc.subcore_barrier()

    def body(j, carry):
        pltpu.sync_copy(onesv, degs.at[dstv.at[j]], add=True)
        return carry

    lax.fori_loop(0, C, body, 0)
    plsc.subcore_barrier()
    pltpu.sync_copy(
        degs.at[pl.ds(si * RPT, RPT)],
        deg_hbm.at[pl.ds(ci * N_PAD + si * RPT, RPT)],
    )


def _make_sc_agg(D):
    DH = D // 2

    @functools.partial(
        pl.kernel,
        out_type=jax.ShapeDtypeStruct((NC * N_PAD, DH), jnp.float32),
        mesh=_mesh,
        scratch_types=[
            pltpu.VMEM((CTA, K), jnp.int32),
            pltpu.VMEM((1, K), jnp.int32),
            pltpu.VMEM((1, K), jnp.int32),
            pltpu.VMEM((K, DH), jnp.float32),
            pltpu.VMEM((K, DH), jnp.float32),
            pltpu.VMEM((16, DH), jnp.float32),
            pltpu.VMEM_SHARED((N_PAD, DH), jnp.float32),
            pltpu.VMEM_SHARED((N_PAD, DH), jnp.float32),
            pltpu.SemaphoreType.DMA,
            pltpu.SemaphoreType.DMA,
        ],
        compiler_params=pltpu.CompilerParams(use_tc_tiling_on_sc=False),
    )
    def agg(glo_hbm, ghi_hbm, src_hbm, dst_hbm, out_hbm, srcv, dstb0, dstb1,
            rows0, rows1, zv, gsh, accs, gsem, dsem):
        ci = lax.axis_index("c")
        si = lax.axis_index("s")
        for r in range(16):
            for c5 in range(DH // 16):
                zv[r, pl.ds(c5 * 16, 16)] = jnp.zeros((16,), jnp.float32)
        for t in range(RPT // 16):
            pltpu.sync_copy(zv, accs.at[pl.ds(si * RPT + t * 16, 16)])

        @pl.when(ci == 0)
        def _():
            pltpu.sync_copy(glo_hbm.at[pl.ds(si * RPT, RPT)],
                            gsh.at[pl.ds(si * RPT, RPT)])

        @pl.when(ci != 0)
        def _():
            pltpu.sync_copy(ghi_hbm.at[pl.ds(si * RPT, RPT)],
                            gsh.at[pl.ds(si * RPT, RPT)])

        pltpu.sync_copy(src_hbm.at[si], srcv)
        plsc.subcore_barrier()

        pltpu.async_copy(gsh.at[srcv.at[0]], rows0, gsem)
        pltpu.async_copy(dst_hbm.at[si, 0], dstb0.at[0], dsem)

        def body(i, carry):
            j = i * 2
            pltpu.make_async_copy(gsh.at[srcv.at[j]], rows0, gsem).wait()
            pltpu.async_copy(gsh.at[srcv.at[j + 1]], rows1, gsem)
            pltpu.make_async_copy(dst_hbm.at[si, j], dstb0.at[0], dsem).wait()
            pltpu.async_copy(dst_hbm.at[si, j + 1], dstb1.at[0], dsem)
            pltpu.sync_copy(rows0, accs.at[dstb0.at[0]], add=True)
            pltpu.make_async_copy(gsh.at[srcv.at[j + 1]], rows1, gsem).wait()
            pltpu.async_copy(gsh.at[srcv.at[j + 2]], rows0, gsem)
            pltpu.make_async_copy(dst_hbm.at[si, j + 1], dstb1.at[0], dsem).wait()
            pltpu.async_copy(dst_hbm.at[si, j + 2], dstb0.at[0], dsem)
            pltpu.sync_copy(rows1, accs.at[dstb1.at[0]], add=True)
            return carry

        lax.fori_loop(0, (CTA - 1) // 2, body, 0)
        pltpu.make_async_copy(gsh.at[srcv.at[CTA - 1]], rows0, gsem).wait()
        pltpu.make_async_copy(dst_hbm.at[si, CTA - 1], dstb0.at[0], dsem).wait()
        pltpu.sync_copy(rows0, accs.at[dstb0.at[0]], add=True)
        plsc.subcore_barrier()
        base = si * RPT
        pltpu.sync_copy(
            accs.at[pl.ds(base, RPT)],
            out_hbm.at[pl.ds(ci * N_PAD + base, RPT)],
        )

    return agg


_sc_agg_hid = _make_sc_agg(DH_PAD)
_sc_agg_out = _make_sc_agg(D_OUT)


def _tc_lin1_body(x_ref, w_ref, deg_ref, glo_ref, ghi_ref):
    dg = deg_ref[:, 0] + deg_ref[:, 1] + 1.0
    dinv = lax.rsqrt(dg)[:, None]
    h = jnp.dot(x_ref[:], w_ref[:], preferred_element_type=jnp.float32)
    g = h * dinv
    glo_ref[:] = g[:, :DH_PAD // 2]
    ghi_ref[:] = g[:, DH_PAD // 2:]


def _tc_mid_body(alo_ref, ahi_ref, glo_ref, ghi_ref, deg_ref, b1_ref, w2_ref,
                 g2lo_ref, g2hi_ref):
    dg = deg_ref[:, 0] + deg_ref[:, 1] + 1.0
    dinv = lax.rsqrt(dg)[:, None]
    zfull = jnp.concatenate(
        [alo_ref[:] + glo_ref[:], ahi_ref[:] + ghi_ref[:]], axis=1)
    z = dinv * zfull + b1_ref[:]
    r = jnp.maximum(z, 0.0)
    h2 = jnp.dot(r, w2_ref[:], preferred_element_type=jnp.float32)
    g2 = h2 * dinv
    g2lo_ref[:] = g2[:, :D_OUT // 2]
    g2hi_ref[:] = g2[:, D_OUT // 2:]


def _tc_out_body(alo_ref, ahi_ref, glo_ref, ghi_ref, deg_ref, b2_ref, out_ref):
    dg = deg_ref[:, 0] + deg_ref[:, 1] + 1.0
    dinv = lax.rsqrt(dg)[:, None]
    zfull = jnp.concatenate(
        [alo_ref[:] + glo_ref[:], ahi_ref[:] + ghi_ref[:]], axis=1)
    z = dinv * zfull + b2_ref[:]
    m = jnp.max(z, axis=1, keepdims=True)
    lse = m + jnp.log(jnp.sum(jnp.exp(z - m), axis=1, keepdims=True))
    out_ref[:] = z - lse


def _tc_lin1(x_pad, w1p, deg2t):
    hh = DH_PAD // 2
    return pl.pallas_call(
        _tc_lin1_body,
        grid=(N_PAD // BLK,),
        in_specs=[
            pl.BlockSpec((BLK, D_IN), lambda i: (i, 0)),
            pl.BlockSpec((D_IN, DH_PAD), lambda i: (0, 0)),
            pl.BlockSpec((BLK, NC), lambda i: (i, 0)),
        ],
        out_specs=[pl.BlockSpec((BLK, hh), lambda i: (i, 0)),
                   pl.BlockSpec((BLK, hh), lambda i: (i, 0))],
        out_shape=[jax.ShapeDtypeStruct((N_PAD, hh), jnp.float32),
                   jax.ShapeDtypeStruct((N_PAD, hh), jnp.float32)],
    )(x_pad, w1p, deg2t)


def _tc_mid(alo, ahi, g1lo, g1hi, deg2t, b1p, w2p):
    hh = DH_PAD // 2
    oh = D_OUT // 2
    return pl.pallas_call(
        _tc_mid_body,
        grid=(N_PAD // BLK,),
        in_specs=[
            pl.BlockSpec((BLK, hh), lambda i: (i, 0)),
            pl.BlockSpec((BLK, hh), lambda i: (i, 0)),
            pl.BlockSpec((BLK, hh), lambda i: (i, 0)),
            pl.BlockSpec((BLK, hh), lambda i: (i, 0)),
            pl.BlockSpec((BLK, NC), lambda i: (i, 0)),
            pl.BlockSpec((1, DH_PAD), lambda i: (0, 0)),
            pl.BlockSpec((DH_PAD, D_OUT), lambda i: (0, 0)),
        ],
        out_specs=[pl.BlockSpec((BLK, oh), lambda i: (i, 0)),
                   pl.BlockSpec((BLK, oh), lambda i: (i, 0))],
        out_shape=[jax.ShapeDtypeStruct((N_PAD, oh), jnp.float32),
                   jax.ShapeDtypeStruct((N_PAD, oh), jnp.float32)],
    )(alo, ahi, g1lo, g1hi, deg2t, b1p, w2p)


def _tc_out(alo, ahi, g2lo, g2hi, deg2t, b2r):
    oh = D_OUT // 2
    return pl.pallas_call(
        _tc_out_body,
        grid=(N_PAD // BLK,),
        in_specs=[
            pl.BlockSpec((BLK, oh), lambda i: (i, 0)),
            pl.BlockSpec((BLK, oh), lambda i: (i, 0)),
            pl.BlockSpec((BLK, oh), lambda i: (i, 0)),
            pl.BlockSpec((BLK, oh), lambda i: (i, 0)),
            pl.BlockSpec((BLK, NC), lambda i: (i, 0)),
            pl.BlockSpec((1, D_OUT), lambda i: (0, 0)),
        ],
        out_specs=pl.BlockSpec((BLK, D_OUT), lambda i: (i, 0)),
        out_shape=jax.ShapeDtypeStruct((N_PAD, D_OUT), jnp.float32),
    )(alo, ahi, g2lo, g2hi, deg2t, b2r)


@jax.jit
def _run(x, edge_index, W1, b1, W2, b2):
    x_pad = jnp.zeros((N_PAD, D_IN), jnp.float32).at[:N].set(x)
    padd = jnp.full((E_PAD - E,), N, jnp.int32)
    pada = jnp.full((E_PAD_A - E,), N, jnp.int32)
    dst_deg = jnp.concatenate([edge_index[1], padd]).reshape(NW, C, K)
    src = jnp.concatenate([edge_index[0], pada]).reshape(NS, CTA, K)
    dst = jnp.concatenate([edge_index[1], pada]).reshape(NS, CTA, K)
    w1p = jnp.zeros((D_IN, DH_PAD), jnp.float32).at[:, :D_HID].set(W1)
    b1p = jnp.zeros((1, DH_PAD), jnp.float32).at[0, :D_HID].set(b1)
    w2p = jnp.zeros((DH_PAD, D_OUT), jnp.float32).at[:D_HID].set(W2)
    b2r = b2.reshape(1, D_OUT)

    degf = _sc_deg(dst_deg)
    deg2t = degf.reshape(NC, N_PAD).T

    g1lo, g1hi = _tc_lin1(x_pad, w1p, deg2t)
    acc1 = _sc_agg_hid(g1lo, g1hi, src, dst)
    g2lo, g2hi = _tc_mid(acc1[:N_PAD], acc1[N_PAD:], g1lo, g1hi, deg2t,
                         b1p, w2p)
    acc2 = _sc_agg_out(g2lo, g2hi, src, dst)
    out = _tc_out(acc2[:N_PAD], acc2[N_PAD:], g2lo, g2hi, deg2t, b2r)
    return out[:N]


def kernel(x, edge_index, W1, b1, W2, b2):
    return _run(x, edge_index, W1, b1, W2, b2)
